# Initial kernel scaffold; baseline (speedup 1.0000x reference)
import jax
import jax.numpy as jnp
from jax.experimental import pallas as pl


def kernel(x_a, x_b, edge_index_ab, edge_index_ba, batch_a, batch_b, params):
    raise NotImplementedError("write your pallas kernel here")


# speedup vs baseline: 4.1003x; 4.1003x over previous
import functools

import jax
import jax.numpy as jnp
from jax import lax
from jax.experimental import pallas as pl
from jax.experimental.pallas import tpu as pltpu
from jax.experimental.pallas import tpu_sc as plsc

H = 128
N = 10000
E = 320000
G = 64
NT = 16
CHUNK = 128
CPT = 160
E_PAD = CPT * CHUNK * NT
ACC_ROWS = N + 240
LANE = 16

_MESH = plsc.VectorSubcoreMesh(core_axis_name="c", subcore_axis_name="s")


@functools.partial(
    pl.kernel,
    mesh=_MESH,
    out_type=[
        jax.ShapeDtypeStruct((N, H), jnp.float32),
        jax.ShapeDtypeStruct((N, H), jnp.float32),
    ],
    scratch_types=[
        pltpu.VMEM_SHARED((ACC_ROWS, H), jnp.float32),
        pltpu.VMEM((16, CHUNK), jnp.int32),
        pltpu.VMEM((16, CHUNK), jnp.int32),
        pltpu.VMEM((CHUNK, H), jnp.float32),
        pltpu.SemaphoreType.DMA,
    ],
)
def _hetero_agg(h_a, h_b, src_ab, dst_ab, src_ba, dst_ba,
                agg_b_out, agg_a_out,
                acc, src_v, dst_v, rows_v, sem):
    c = lax.axis_index("c")
    s = lax.axis_index("s")

    def _zrow(i, _):
        def _zcol(j, _):
            rows_v[i, pl.ds(j * LANE, LANE)] = jnp.zeros((LANE,), jnp.float32)
            return 0
        return lax.fori_loop(0, H // LANE, _zcol, 0)
    lax.fori_loop(0, CHUNK, _zrow, 0)

    def _zacc(k, _):
        pltpu.sync_copy(rows_v, acc.at[pl.ds((s * 5 + k) * 128, 128)])
        return 0
    lax.fori_loop(0, ACC_ROWS // (NT * 128), _zacc, 0)
    plsc.subcore_barrier()

    def _run(h, src2, dst2):
        base = s * CPT

        def _blk(o, _):
            pltpu.sync_copy(src2.at[pl.ds(base + o * 16, 16)], src_v)
            pltpu.sync_copy(dst2.at[pl.ds(base + o * 16, 16)], dst_v)

            def _step(j, _):
                pltpu.async_copy(h.at[src_v.at[j]], rows_v, sem).wait()
                pltpu.sync_copy(rows_v, acc.at[dst_v.at[j]], add=True)
                return 0
            return lax.fori_loop(0, 16, _step, 0)
        lax.fori_loop(0, CPT // 16, _blk, 0)

    @pl.when(c == 0)
    def _():
        _run(h_a, src_ab, dst_ab)

    @pl.when(c != 0)
    def _():
        _run(h_b, src_ba, dst_ba)

    plsc.subcore_barrier()

    def _copy_out(out):
        @pl.when(s < 15)
        def _():
            pltpu.sync_copy(acc.at[pl.ds(s * 640, 640)],
                            out.at[pl.ds(s * 640, 640)])

        @pl.when(s == 15)
        def _():
            pltpu.sync_copy(acc.at[pl.ds(9600, 400)],
                            out.at[pl.ds(9600, 400)])

    @pl.when(c == 0)
    def _():
        _copy_out(agg_b_out)

    @pl.when(c != 0)
    def _():
        _copy_out(agg_a_out)


N_PAD_POOL = 16384
PCH = N_PAD_POOL // (NT * CHUNK)
TAB_ROWS = G + 8


@functools.partial(
    pl.kernel,
    mesh=_MESH,
    out_type=[
        jax.ShapeDtypeStruct((G, H), jnp.float32),
        jax.ShapeDtypeStruct((G, H), jnp.float32),
    ],
    scratch_types=[
        pltpu.VMEM_SHARED((NT, G, H), jnp.float32),
        pltpu.VMEM((PCH, CHUNK), jnp.int32),
        pltpu.VMEM((PCH, CHUNK), jnp.int32),
        pltpu.VMEM((CHUNK, H), jnp.float32),
        pltpu.VMEM((TAB_ROWS, H), jnp.float32),
        pltpu.VMEM((NT, 8, H), jnp.float32),
        pltpu.SemaphoreType.DMA,
    ],
)
def _pool(h_a, h_b, batch_a2, batch_b2, rowidx2,
          ga_out, gb_out,
          part, gidx_v, bat_v, rows_v, tab_v, mrg_v, sem):
    c = lax.axis_index("c")
    s = lax.axis_index("s")
    neg_inf = jnp.full((LANE,), -jnp.inf, jnp.float32)

    def _irow(i, _):
        def _icol(j, _):
            tab_v[i, pl.ds(j * LANE, LANE)] = neg_inf
            return 0
        return lax.fori_loop(0, H // LANE, _icol, 0)
    lax.fori_loop(0, TAB_ROWS, _irow, 0)

    base = s * PCH
    pltpu.sync_copy(rowidx2.at[pl.ds(base, PCH)], gidx_v)

    def _run(h, bat2):
        pltpu.sync_copy(bat2.at[pl.ds(base, PCH)], bat_v)

        def _chunk(k, _):
            pltpu.async_copy(h.at[gidx_v.at[k]], rows_v, sem).wait()

            def _grp(g16, _):
                bvec = bat_v[k, pl.ds(g16 * LANE, LANE)]
                for ii in range(LANE):
                    bi = bvec[ii]
                    i = g16 * LANE + ii

                    def _col(j, _, bi=bi, i=i):
                        cur = tab_v[bi, pl.ds(j * LANE, LANE)]
                        val = rows_v[i, pl.ds(j * LANE, LANE)]
                        tab_v[bi, pl.ds(j * LANE, LANE)] = jnp.maximum(cur, val)
                        return 0
                    lax.fori_loop(0, H // LANE, _col, 0)
                return 0
            return lax.fori_loop(0, CHUNK // LANE, _grp, 0)
        lax.fori_loop(0, PCH, _chunk, 0)

    @pl.when(c == 0)
    def _():
        _run(h_a, batch_a2)

    @pl.when(c != 0)
    def _():
        _run(h_b, batch_b2)

    pltpu.sync_copy(tab_v.at[pl.ds(0, G)], part.at[s])
    plsc.subcore_barrier()

    @pl.when(s < 8)
    def _():
        def _fetch(t, _):
            pltpu.sync_copy(part.at[t, pl.ds(s * 8, 8)], mrg_v.at[t])
            return 0
        lax.fori_loop(0, NT, _fetch, 0)

        def _red_t(t, _):
            def _red_g(g, _):
                def _red_j(j, _):
                    a = mrg_v[0, g, pl.ds(j * LANE, LANE)]
                    b = mrg_v[t, g, pl.ds(j * LANE, LANE)]
                    mrg_v[0, g, pl.ds(j * LANE, LANE)] = jnp.maximum(a, b)
                    return 0
                return lax.fori_loop(0, H // LANE, _red_j, 0)
            return lax.fori_loop(0, 8, _red_g, 0)
        lax.fori_loop(1, NT, _red_t, 0)

        @pl.when(c == 0)
        def _():
            pltpu.sync_copy(mrg_v.at[0], ga_out.at[pl.ds(s * 8, 8)])

        @pl.when(c != 0)
        def _():
            pltpu.sync_copy(mrg_v.at[0], gb_out.at[pl.ds(s * 8, 8)])


def _dense_body(relu, agg_ref, h_ref, wrel_ref, wroot_ref, b_ref, o_ref):
    y = (jnp.dot(agg_ref[...], wrel_ref[...], preferred_element_type=jnp.float32)
         + jnp.dot(h_ref[...], wroot_ref[...], preferred_element_type=jnp.float32)
         + b_ref[...])
    o_ref[...] = jnp.maximum(y, 0.0) if relu else y


def _dense(agg, h, wrel, wroot, b, relu):
    B = 1000
    return pl.pallas_call(
        functools.partial(_dense_body, relu),
        grid=(N // B,),
        in_specs=[
            pl.BlockSpec((B, H), lambda i: (i, 0)),
            pl.BlockSpec((B, H), lambda i: (i, 0)),
            pl.BlockSpec((H, H), lambda i: (0, 0)),
            pl.BlockSpec((H, H), lambda i: (0, 0)),
            pl.BlockSpec((1, H), lambda i: (0, 0)),
        ],
        out_specs=pl.BlockSpec((B, H), lambda i: (i, 0)),
        out_shape=jax.ShapeDtypeStruct((N, H), jnp.float32),
    )(agg, h, wrel, wroot, b.reshape(1, H))


def _mlp_body(ga_ref, gb_ref, w1a, b1a, w2a, b2a, w1b, b1b, w2b, b2b,
              ow, obias, o_ref):
    oa = jnp.maximum(
        jnp.dot(ga_ref[...], w1a[...], preferred_element_type=jnp.float32)
        + b1a[...], 0.0)
    oa = jnp.dot(oa, w2a[...], preferred_element_type=jnp.float32) + b2a[...]
    ob = jnp.maximum(
        jnp.dot(gb_ref[...], w1b[...], preferred_element_type=jnp.float32)
        + b1b[...], 0.0)
    ob = jnp.dot(ob, w2b[...], preferred_element_type=jnp.float32) + b2b[...]
    o_ref[...] = oa * ow[0, 0] + ob * ow[1, 0] + obias[0, 0]


def _mlp(ga, gb, p):
    args = (ga, gb,
            p["mlpW1_a"], p["mlpb1_a"].reshape(1, 5),
            p["mlpW2_a"], p["mlpb2_a"].reshape(1, 1),
            p["mlpW1_b"], p["mlpb1_b"].reshape(1, 5),
            p["mlpW2_b"], p["mlpb2_b"].reshape(1, 1),
            p["outW"], p["outb"].reshape(1, 1))
    return pl.pallas_call(
        _mlp_body,
        out_shape=jax.ShapeDtypeStruct((G, 1), jnp.float32),
    )(*args)


def _prep_edges(ei):
    pad = E_PAD - E
    src_pad = (jnp.arange(pad, dtype=jnp.int32) * 97) % N
    dst_pad = N + (jnp.arange(pad, dtype=jnp.int32) % 128)
    src = jnp.concatenate([ei[0], src_pad]).reshape(E_PAD // CHUNK, CHUNK)
    dst = jnp.concatenate([ei[1], dst_pad]).reshape(E_PAD // CHUNK, CHUNK)
    return src, dst


def kernel(x_a, x_b, edge_index_ab, edge_index_ba, batch_a, batch_b, params):
    src_ab, dst_ab = _prep_edges(edge_index_ab)
    src_ba, dst_ba = _prep_edges(edge_index_ba)

    pad_n = N_PAD_POOL - N
    batch_a2 = jnp.concatenate(
        [batch_a, jnp.full((pad_n,), G, jnp.int32)]).reshape(-1, CHUNK)
    batch_b2 = jnp.concatenate(
        [batch_b, jnp.full((pad_n,), G, jnp.int32)]).reshape(-1, CHUNK)
    rowidx2 = jnp.minimum(jnp.arange(N_PAD_POOL, dtype=jnp.int32),
                          N - 1).reshape(-1, CHUNK)

    h_a, h_b = x_a, x_b
    for l in range(3):
        agg_b, agg_a = _hetero_agg(h_a, h_b, src_ab, dst_ab, src_ba, dst_ba)
        relu = l < 2
        new_b = _dense(agg_b, h_b, params[f"Wrel_{l}_ab"],
                       params[f"Wroot_{l}_ab"], params[f"brel_{l}_ab"], relu)
        new_a = _dense(agg_a, h_a, params[f"Wrel_{l}_ba"],
                       params[f"Wroot_{l}_ba"], params[f"brel_{l}_ba"], relu)
        h_a, h_b = new_a, new_b

    ga, gb = _pool(h_a, h_b, batch_a2, batch_b2, rowidx2)
    return _mlp(ga, gb, params)


# speedup vs baseline: 7.2735x; 1.7739x over previous
import functools

import jax
import jax.numpy as jnp
from jax import lax
from jax.experimental import pallas as pl
from jax.experimental.pallas import tpu as pltpu
from jax.experimental.pallas import tpu_sc as plsc

H = 128
N = 10000
E = 320000
G = 64
NT = 16
CHUNK = 128
CPT = 160
E_PAD = CPT * CHUNK * NT
ACC_ROWS = N + 240
LANE = 16

_MESH = plsc.VectorSubcoreMesh(core_axis_name="c", subcore_axis_name="s")


@functools.partial(
    pl.kernel,
    mesh=_MESH,
    out_type=[
        jax.ShapeDtypeStruct((N, H), jnp.float32),
        jax.ShapeDtypeStruct((N, H), jnp.float32),
    ],
    scratch_types=[
        pltpu.VMEM_SHARED((ACC_ROWS, H), jnp.float32),
        pltpu.VMEM((16, CHUNK), jnp.int32),
        pltpu.VMEM((16, CHUNK), jnp.int32),
        pltpu.VMEM((CHUNK, H), jnp.float32),
        pltpu.VMEM((CHUNK, H), jnp.float32),
        pltpu.SemaphoreType.DMA,
        pltpu.SemaphoreType.DMA,
    ],
)
def _hetero_agg(h_a, h_b, src_ab, dst_ab, src_ba, dst_ba,
                agg_b_out, agg_a_out,
                acc, src_v, dst_v, rows0_v, rows1_v, sem0, sem1):
    c = lax.axis_index("c")
    s = lax.axis_index("s")

    def _zrow(i, _):
        def _zcol(j, _):
            rows0_v[i, pl.ds(j * LANE, LANE)] = jnp.zeros((LANE,), jnp.float32)
            return 0
        return lax.fori_loop(0, H // LANE, _zcol, 0)
    lax.fori_loop(0, CHUNK, _zrow, 0)

    def _zacc(k, _):
        pltpu.sync_copy(rows0_v, acc.at[pl.ds((s * 5 + k) * 128, 128)])
        return 0
    lax.fori_loop(0, ACC_ROWS // (NT * 128), _zacc, 0)
    plsc.subcore_barrier()

    def _run(h, src2, dst2):
        base = s * CPT

        def _blk(o, _):
            pltpu.sync_copy(src2.at[pl.ds(base + o * 16, 16)], src_v)
            pltpu.sync_copy(dst2.at[pl.ds(base + o * 16, 16)], dst_v)
            pltpu.async_copy(h.at[src_v.at[0]], rows0_v, sem0)

            def _pair(p, _):
                pltpu.async_copy(h.at[src_v.at[2 * p + 1]], rows1_v, sem1)
                pltpu.make_async_copy(h.at[src_v.at[2 * p]], rows0_v,
                                      sem0).wait()
                pltpu.sync_copy(rows0_v, acc.at[dst_v.at[2 * p]], add=True)

                @pl.when(p < 7)
                def _():
                    pltpu.async_copy(h.at[src_v.at[2 * p + 2]], rows0_v, sem0)
                pltpu.make_async_copy(h.at[src_v.at[2 * p + 1]], rows1_v,
                                      sem1).wait()
                pltpu.sync_copy(rows1_v, acc.at[dst_v.at[2 * p + 1]], add=True)
                return 0
            return lax.fori_loop(0, 8, _pair, 0)
        lax.fori_loop(0, CPT // 16, _blk, 0)

    @pl.when(c == 0)
    def _():
        _run(h_a, src_ab, dst_ab)

    @pl.when(c != 0)
    def _():
        _run(h_b, src_ba, dst_ba)

    plsc.subcore_barrier()

    def _copy_out(out):
        @pl.when(s < 15)
        def _():
            pltpu.sync_copy(acc.at[pl.ds(s * 640, 640)],
                            out.at[pl.ds(s * 640, 640)])

        @pl.when(s == 15)
        def _():
            pltpu.sync_copy(acc.at[pl.ds(9600, 400)],
                            out.at[pl.ds(9600, 400)])

    @pl.when(c == 0)
    def _():
        _copy_out(agg_b_out)

    @pl.when(c != 0)
    def _():
        _copy_out(agg_a_out)


ROWS_PT = 640
N_PAD_POOL = NT * ROWS_PT
TAB_ROWS = G + 8


@functools.partial(
    pl.kernel,
    mesh=_MESH,
    out_type=[
        jax.ShapeDtypeStruct((G, H), jnp.float32),
        jax.ShapeDtypeStruct((G, H), jnp.float32),
    ],
    scratch_types=[
        pltpu.VMEM_SHARED((NT, G, H), jnp.float32),
        pltpu.VMEM((ROWS_PT // CHUNK, CHUNK), jnp.int32),
        pltpu.VMEM((ROWS_PT, H), jnp.float32),
        pltpu.VMEM((TAB_ROWS, H), jnp.float32),
        pltpu.VMEM((NT, 8, H), jnp.float32),
    ],
)
def _pool(h_a, h_b, batch_a3, batch_b3,
          ga_out, gb_out,
          part, bat_v, rows_v, tab_v, mrg_v):
    c = lax.axis_index("c")
    s = lax.axis_index("s")
    neg_inf = jnp.full((LANE,), -jnp.inf, jnp.float32)

    def _irow(i, _):
        def _icol(j, _):
            tab_v[i, pl.ds(j * LANE, LANE)] = neg_inf
            return 0
        return lax.fori_loop(0, H // LANE, _icol, 0)
    lax.fori_loop(0, TAB_ROWS, _irow, 0)

    def _run(h, bat3):
        pltpu.sync_copy(bat3.at[s], bat_v)
        @pl.when(s < 15)
        def _():
            pltpu.sync_copy(h.at[pl.ds(s * ROWS_PT, ROWS_PT)], rows_v)

        @pl.when(s == 15)
        def _():
            pltpu.sync_copy(h.at[pl.ds(15 * ROWS_PT, N - 15 * ROWS_PT)],
                            rows_v.at[pl.ds(0, N - 15 * ROWS_PT)])

        def _grp(g, carry):
            prev_bi = carry[0]
            runs = carry[1:]
            bvec = bat_v[g // 8, pl.ds((g % 8) * LANE, LANE)]
            for ii in range(LANE):
                bi = bvec[ii]
                i = g * LANE + ii
                changed = jnp.logical_and(bi != prev_bi, prev_bi >= 0)

                @pl.when(changed)
                def _(runs=runs, prev_bi=prev_bi):
                    for j in range(H // LANE):
                        cur = tab_v[prev_bi, pl.ds(j * LANE, LANE)]
                        tab_v[prev_bi, pl.ds(j * LANE, LANE)] = \
                            jnp.maximum(cur, runs[j])

                fresh = jnp.logical_or(changed, prev_bi < 0)
                runs = tuple(
                    jnp.maximum(jnp.where(fresh, neg_inf, runs[j]),
                                rows_v[i, pl.ds(j * LANE, LANE)])
                    for j in range(H // LANE))
                prev_bi = bi
            return (prev_bi,) + runs

        init = (jnp.int32(-1),) + tuple(neg_inf for _ in range(H // LANE))
        final = lax.fori_loop(0, ROWS_PT // LANE, _grp, init)
        last_bi = final[0]

        @pl.when(last_bi >= 0)
        def _():
            for j in range(H // LANE):
                cur = tab_v[last_bi, pl.ds(j * LANE, LANE)]
                tab_v[last_bi, pl.ds(j * LANE, LANE)] = \
                    jnp.maximum(cur, final[1 + j])

    @pl.when(c == 0)
    def _():
        _run(h_a, batch_a3)

    @pl.when(c != 0)
    def _():
        _run(h_b, batch_b3)

    pltpu.sync_copy(tab_v.at[pl.ds(0, G)], part.at[s])
    plsc.subcore_barrier()

    @pl.when(s < 8)
    def _():
        def _fetch(t, _):
            pltpu.sync_copy(part.at[t, pl.ds(s * 8, 8)], mrg_v.at[t])
            return 0
        lax.fori_loop(0, NT, _fetch, 0)

        def _red_t(t, _):
            def _red_g(g, _):
                def _red_j(j, _):
                    a = mrg_v[0, g, pl.ds(j * LANE, LANE)]
                    b = mrg_v[t, g, pl.ds(j * LANE, LANE)]
                    mrg_v[0, g, pl.ds(j * LANE, LANE)] = jnp.maximum(a, b)
                    return 0
                return lax.fori_loop(0, H // LANE, _red_j, 0)
            return lax.fori_loop(0, 8, _red_g, 0)
        lax.fori_loop(1, NT, _red_t, 0)

        @pl.when(c == 0)
        def _():
            pltpu.sync_copy(mrg_v.at[0], ga_out.at[pl.ds(s * 8, 8)])

        @pl.when(c != 0)
        def _():
            pltpu.sync_copy(mrg_v.at[0], gb_out.at[pl.ds(s * 8, 8)])


def _dense_body(relu, agg_ref, h_ref, wrel_ref, wroot_ref, b_ref, o_ref):
    y = (jnp.dot(agg_ref[...], wrel_ref[...], preferred_element_type=jnp.float32)
         + jnp.dot(h_ref[...], wroot_ref[...], preferred_element_type=jnp.float32)
         + b_ref[...])
    o_ref[...] = jnp.maximum(y, 0.0) if relu else y


def _dense(agg, h, wrel, wroot, b, relu):
    B = 1000
    return pl.pallas_call(
        functools.partial(_dense_body, relu),
        grid=(N // B,),
        in_specs=[
            pl.BlockSpec((B, H), lambda i: (i, 0)),
            pl.BlockSpec((B, H), lambda i: (i, 0)),
            pl.BlockSpec((H, H), lambda i: (0, 0)),
            pl.BlockSpec((H, H), lambda i: (0, 0)),
            pl.BlockSpec((1, H), lambda i: (0, 0)),
        ],
        out_specs=pl.BlockSpec((B, H), lambda i: (i, 0)),
        out_shape=jax.ShapeDtypeStruct((N, H), jnp.float32),
    )(agg, h, wrel, wroot, b.reshape(1, H))


def _mlp_body(ga_ref, gb_ref, w1a, b1a, w2a, b2a, w1b, b1b, w2b, b2b,
              ow, obias, o_ref):
    oa = jnp.maximum(
        jnp.dot(ga_ref[...], w1a[...], preferred_element_type=jnp.float32)
        + b1a[...], 0.0)
    oa = jnp.dot(oa, w2a[...], preferred_element_type=jnp.float32) + b2a[...]
    ob = jnp.maximum(
        jnp.dot(gb_ref[...], w1b[...], preferred_element_type=jnp.float32)
        + b1b[...], 0.0)
    ob = jnp.dot(ob, w2b[...], preferred_element_type=jnp.float32) + b2b[...]
    o_ref[...] = oa * ow[0, 0] + ob * ow[1, 0] + obias[0, 0]


def _mlp(ga, gb, p):
    args = (ga, gb,
            p["mlpW1_a"], p["mlpb1_a"].reshape(1, 5),
            p["mlpW2_a"], p["mlpb2_a"].reshape(1, 1),
            p["mlpW1_b"], p["mlpb1_b"].reshape(1, 5),
            p["mlpW2_b"], p["mlpb2_b"].reshape(1, 1),
            p["outW"], p["outb"].reshape(1, 1))
    return pl.pallas_call(
        _mlp_body,
        out_shape=jax.ShapeDtypeStruct((G, 1), jnp.float32),
    )(*args)


def _prep_edges(ei):
    pad = E_PAD - E
    src_pad = (jnp.arange(pad, dtype=jnp.int32) * 97) % N
    dst_pad = N + (jnp.arange(pad, dtype=jnp.int32) % 128)
    src = jnp.concatenate([ei[0], src_pad]).reshape(E_PAD // CHUNK, CHUNK)
    dst = jnp.concatenate([ei[1], dst_pad]).reshape(E_PAD // CHUNK, CHUNK)
    return src, dst


def kernel(x_a, x_b, edge_index_ab, edge_index_ba, batch_a, batch_b, params):
    src_ab, dst_ab = _prep_edges(edge_index_ab)
    src_ba, dst_ba = _prep_edges(edge_index_ba)

    pad_n = N_PAD_POOL - N
    batch_a3 = jnp.concatenate(
        [batch_a, jnp.full((pad_n,), G, jnp.int32)]).reshape(NT, -1, CHUNK)
    batch_b3 = jnp.concatenate(
        [batch_b, jnp.full((pad_n,), G, jnp.int32)]).reshape(NT, -1, CHUNK)

    h_a, h_b = x_a, x_b
    for l in range(3):
        agg_b, agg_a = _hetero_agg(h_a, h_b, src_ab, dst_ab, src_ba, dst_ba)
        relu = l < 2
        new_b = _dense(agg_b, h_b, params[f"Wrel_{l}_ab"],
                       params[f"Wroot_{l}_ab"], params[f"brel_{l}_ab"], relu)
        new_a = _dense(agg_a, h_a, params[f"Wrel_{l}_ba"],
                       params[f"Wroot_{l}_ba"], params[f"brel_{l}_ba"], relu)
        h_a, h_b = new_a, new_b

    ga, gb = _pool(h_a, h_b, batch_a3, batch_b3)
    return _mlp(ga, gb, params)


# speedup vs baseline: 7.6012x; 1.0451x over previous
import functools

import jax
import jax.numpy as jnp
from jax import lax
from jax.experimental import pallas as pl
from jax.experimental.pallas import tpu as pltpu
from jax.experimental.pallas import tpu_sc as plsc

H = 128
N = 10000
E = 320000
G = 64
NT = 16
CHUNK = 128
CPT = 160
E_PAD = CPT * CHUNK * NT
ACC_ROWS = N + 240
LANE = 16

_MESH = plsc.VectorSubcoreMesh(core_axis_name="c", subcore_axis_name="s")


@functools.partial(
    pl.kernel,
    mesh=_MESH,
    out_type=[
        jax.ShapeDtypeStruct((2, N, H), jnp.float32),
    ],
    scratch_types=[
        pltpu.VMEM_SHARED((ACC_ROWS, H), jnp.float32),
        pltpu.VMEM((32, CHUNK), jnp.int32),
        pltpu.VMEM((32, CHUNK), jnp.int32),
        pltpu.VMEM((CHUNK, H), jnp.float32),
        pltpu.VMEM((CHUNK, H), jnp.float32),
        pltpu.SemaphoreType.DMA,
        pltpu.SemaphoreType.DMA,
        pltpu.SemaphoreType.DMA,
    ],
)
def _hetero_agg(hst, src_ab, dst_ab, src_ba, dst_ba,
                agg_out,
                acc, src_v, dst_v, rows0_v, rows1_v, sem0, sem1, sem_i):
    c = lax.axis_index("c")
    s = lax.axis_index("s")

    def _zrow(i, _):
        def _zcol(j, _):
            rows0_v[i, pl.ds(j * LANE, LANE)] = jnp.zeros((LANE,), jnp.float32)
            return 0
        return lax.fori_loop(0, H // LANE, _zcol, 0)
    lax.fori_loop(0, CHUNK, _zrow, 0)

    def _zacc(k, _):
        pltpu.sync_copy(rows0_v, acc.at[pl.ds((s * 5 + k) * 128, 128)])
        return 0
    lax.fori_loop(0, ACC_ROWS // (NT * 128), _zacc, 0)
    plsc.subcore_barrier()

    NBLK = CPT // 16

    def _run(h, src2, dst2):
        base = s * CPT
        pltpu.sync_copy(src2.at[pl.ds(base, 16)], src_v.at[pl.ds(0, 16)])
        pltpu.sync_copy(dst2.at[pl.ds(base, 16)], dst_v.at[pl.ds(0, 16)])

        def _blk(o, _):
            ib = lax.rem(o, 2) * 16
            nb = lax.rem(o + 1, 2) * 16

            @pl.when(o < NBLK - 1)
            def _():
                pltpu.async_copy(src2.at[pl.ds(base + (o + 1) * 16, 16)],
                                 src_v.at[pl.ds(nb, 16)], sem_i)
                pltpu.async_copy(dst2.at[pl.ds(base + (o + 1) * 16, 16)],
                                 dst_v.at[pl.ds(nb, 16)], sem_i)

            pltpu.async_copy(h.at[src_v.at[ib]], rows0_v, sem0)

            def _pair(p, _):
                pltpu.async_copy(h.at[src_v.at[ib + 2 * p + 1]], rows1_v, sem1)
                pltpu.make_async_copy(h.at[src_v.at[ib + 2 * p]], rows0_v,
                                      sem0).wait()
                pltpu.sync_copy(rows0_v, acc.at[dst_v.at[ib + 2 * p]],
                                add=True)

                @pl.when(p < 7)
                def _():
                    pltpu.async_copy(h.at[src_v.at[ib + 2 * p + 2]], rows0_v,
                                     sem0)
                pltpu.make_async_copy(h.at[src_v.at[ib + 2 * p + 1]], rows1_v,
                                      sem1).wait()
                pltpu.sync_copy(rows1_v, acc.at[dst_v.at[ib + 2 * p + 1]],
                                add=True)
                return 0
            lax.fori_loop(0, 8, _pair, 0)

            @pl.when(o < NBLK - 1)
            def _():
                pltpu.make_async_copy(src2.at[pl.ds(base + (o + 1) * 16, 16)],
                                      src_v.at[pl.ds(nb, 16)], sem_i).wait()
                pltpu.make_async_copy(dst2.at[pl.ds(base + (o + 1) * 16, 16)],
                                      dst_v.at[pl.ds(nb, 16)], sem_i).wait()
            return 0
        lax.fori_loop(0, NBLK, _blk, 0)

    @pl.when(c == 0)
    def _():
        _run(hst.at[0], src_ab, dst_ab)

    @pl.when(c != 0)
    def _():
        _run(hst.at[1], src_ba, dst_ba)

    plsc.subcore_barrier()

    def _copy_out(t):
        @pl.when(s < 15)
        def _():
            pltpu.sync_copy(acc.at[pl.ds(s * 640, 640)],
                            agg_out.at[t, pl.ds(s * 640, 640)])

        @pl.when(s == 15)
        def _():
            pltpu.sync_copy(acc.at[pl.ds(9600, 400)],
                            agg_out.at[t, pl.ds(9600, 400)])

    @pl.when(c == 0)
    def _():
        _copy_out(1)

    @pl.when(c != 0)
    def _():
        _copy_out(0)


ROWS_PT = 640
N_PAD_POOL = NT * ROWS_PT
TAB_ROWS = G + 8


@functools.partial(
    pl.kernel,
    mesh=_MESH,
    out_type=[
        jax.ShapeDtypeStruct((G, H), jnp.float32),
        jax.ShapeDtypeStruct((G, H), jnp.float32),
    ],
    scratch_types=[
        pltpu.VMEM_SHARED((NT, G, H), jnp.float32),
        pltpu.VMEM((ROWS_PT // CHUNK, CHUNK), jnp.int32),
        pltpu.VMEM((ROWS_PT, H), jnp.float32),
        pltpu.VMEM((TAB_ROWS, H), jnp.float32),
        pltpu.VMEM((NT, 8, H), jnp.float32),
    ],
)
def _pool(hst, batch_a3, batch_b3,
          ga_out, gb_out,
          part, bat_v, rows_v, tab_v, mrg_v):
    c = lax.axis_index("c")
    s = lax.axis_index("s")
    neg_inf = jnp.full((LANE,), -jnp.inf, jnp.float32)

    def _irow(i, _):
        def _icol(j, _):
            tab_v[i, pl.ds(j * LANE, LANE)] = neg_inf
            return 0
        return lax.fori_loop(0, H // LANE, _icol, 0)
    lax.fori_loop(0, TAB_ROWS, _irow, 0)

    def _run(t, bat3):
        pltpu.sync_copy(bat3.at[s], bat_v)
        @pl.when(s < 15)
        def _():
            pltpu.sync_copy(hst.at[t, pl.ds(s * ROWS_PT, ROWS_PT)], rows_v)

        @pl.when(s == 15)
        def _():
            pltpu.sync_copy(hst.at[t, pl.ds(15 * ROWS_PT, N - 15 * ROWS_PT)],
                            rows_v.at[pl.ds(0, N - 15 * ROWS_PT)])

        def _grp(g, carry):
            prev_bi = carry[0]
            runs = carry[1:]
            bvec = bat_v[g // 8, pl.ds((g % 8) * LANE, LANE)]
            for ii in range(LANE):
                bi = bvec[ii]
                i = g * LANE + ii
                changed = jnp.logical_and(bi != prev_bi, prev_bi >= 0)

                @pl.when(changed)
                def _(runs=runs, prev_bi=prev_bi):
                    for j in range(H // LANE):
                        cur = tab_v[prev_bi, pl.ds(j * LANE, LANE)]
                        tab_v[prev_bi, pl.ds(j * LANE, LANE)] = \
                            jnp.maximum(cur, runs[j])

                fresh = jnp.logical_or(changed, prev_bi < 0)
                runs = tuple(
                    jnp.maximum(jnp.where(fresh, neg_inf, runs[j]),
                                rows_v[i, pl.ds(j * LANE, LANE)])
                    for j in range(H // LANE))
                prev_bi = bi
            return (prev_bi,) + runs

        init = (jnp.int32(-1),) + tuple(neg_inf for _ in range(H // LANE))
        final = lax.fori_loop(0, ROWS_PT // LANE, _grp, init)
        last_bi = final[0]

        @pl.when(last_bi >= 0)
        def _():
            for j in range(H // LANE):
                cur = tab_v[last_bi, pl.ds(j * LANE, LANE)]
                tab_v[last_bi, pl.ds(j * LANE, LANE)] = \
                    jnp.maximum(cur, final[1 + j])

    @pl.when(c == 0)
    def _():
        _run(0, batch_a3)

    @pl.when(c != 0)
    def _():
        _run(1, batch_b3)

    pltpu.sync_copy(tab_v.at[pl.ds(0, G)], part.at[s])
    plsc.subcore_barrier()

    @pl.when(s < 8)
    def _():
        def _fetch(t, _):
            pltpu.sync_copy(part.at[t, pl.ds(s * 8, 8)], mrg_v.at[t])
            return 0
        lax.fori_loop(0, NT, _fetch, 0)

        def _red_t(t, _):
            def _red_g(g, _):
                def _red_j(j, _):
                    a = mrg_v[0, g, pl.ds(j * LANE, LANE)]
                    b = mrg_v[t, g, pl.ds(j * LANE, LANE)]
                    mrg_v[0, g, pl.ds(j * LANE, LANE)] = jnp.maximum(a, b)
                    return 0
                return lax.fori_loop(0, H // LANE, _red_j, 0)
            return lax.fori_loop(0, 8, _red_g, 0)
        lax.fori_loop(1, NT, _red_t, 0)

        @pl.when(c == 0)
        def _():
            pltpu.sync_copy(mrg_v.at[0], ga_out.at[pl.ds(s * 8, 8)])

        @pl.when(c != 0)
        def _():
            pltpu.sync_copy(mrg_v.at[0], gb_out.at[pl.ds(s * 8, 8)])


def _dense_body(relu, agg_ref, h_ref, wrel_ref, wroot_ref, b_ref, o_ref):
    y = (jnp.dot(agg_ref[0], wrel_ref[0], preferred_element_type=jnp.float32)
         + jnp.dot(h_ref[0], wroot_ref[0], preferred_element_type=jnp.float32)
         + b_ref[0])
    o_ref[0] = jnp.maximum(y, 0.0) if relu else y


def _dense(aggst, hst, wrel_st, wroot_st, b_st, relu):
    B = 1000
    return pl.pallas_call(
        functools.partial(_dense_body, relu),
        grid=(2, N // B),
        in_specs=[
            pl.BlockSpec((1, B, H), lambda t, i: (t, i, 0)),
            pl.BlockSpec((1, B, H), lambda t, i: (t, i, 0)),
            pl.BlockSpec((1, H, H), lambda t, i: (t, 0, 0)),
            pl.BlockSpec((1, H, H), lambda t, i: (t, 0, 0)),
            pl.BlockSpec((1, 1, H), lambda t, i: (t, 0, 0)),
        ],
        out_specs=pl.BlockSpec((1, B, H), lambda t, i: (t, i, 0)),
        out_shape=jax.ShapeDtypeStruct((2, N, H), jnp.float32),
    )(aggst, hst, wrel_st, wroot_st, b_st)


def _mlp_body(ga_ref, gb_ref, w1a, b1a, w2a, b2a, w1b, b1b, w2b, b2b,
              ow, obias, o_ref):
    oa = jnp.maximum(
        jnp.dot(ga_ref[...], w1a[...], preferred_element_type=jnp.float32)
        + b1a[...], 0.0)
    oa = jnp.dot(oa, w2a[...], preferred_element_type=jnp.float32) + b2a[...]
    ob = jnp.maximum(
        jnp.dot(gb_ref[...], w1b[...], preferred_element_type=jnp.float32)
        + b1b[...], 0.0)
    ob = jnp.dot(ob, w2b[...], preferred_element_type=jnp.float32) + b2b[...]
    o_ref[...] = oa * ow[0, 0] + ob * ow[1, 0] + obias[0, 0]


def _mlp(ga, gb, p):
    args = (ga, gb,
            p["mlpW1_a"], p["mlpb1_a"].reshape(1, 5),
            p["mlpW2_a"], p["mlpb2_a"].reshape(1, 1),
            p["mlpW1_b"], p["mlpb1_b"].reshape(1, 5),
            p["mlpW2_b"], p["mlpb2_b"].reshape(1, 1),
            p["outW"], p["outb"].reshape(1, 1))
    return pl.pallas_call(
        _mlp_body,
        out_shape=jax.ShapeDtypeStruct((G, 1), jnp.float32),
    )(*args)


def _prep_edges(ei):
    pad = E_PAD - E
    src_pad = (jnp.arange(pad, dtype=jnp.int32) * 97) % N
    dst_pad = N + (jnp.arange(pad, dtype=jnp.int32) % 128)
    src = jnp.concatenate([ei[0], src_pad]).reshape(E_PAD // CHUNK, CHUNK)
    dst = jnp.concatenate([ei[1], dst_pad]).reshape(E_PAD // CHUNK, CHUNK)
    return src, dst


def kernel(x_a, x_b, edge_index_ab, edge_index_ba, batch_a, batch_b, params):
    src_ab, dst_ab = _prep_edges(edge_index_ab)
    src_ba, dst_ba = _prep_edges(edge_index_ba)

    pad_n = N_PAD_POOL - N
    batch_a3 = jnp.concatenate(
        [batch_a, jnp.full((pad_n,), G, jnp.int32)]).reshape(NT, -1, CHUNK)
    batch_b3 = jnp.concatenate(
        [batch_b, jnp.full((pad_n,), G, jnp.int32)]).reshape(NT, -1, CHUNK)

    hst = jnp.stack([x_a, x_b])
    for l in range(3):
        (aggst,) = _hetero_agg(hst, src_ab, dst_ab, src_ba, dst_ba)
        wrel_st = jnp.stack([params[f"Wrel_{l}_ba"], params[f"Wrel_{l}_ab"]])
        wroot_st = jnp.stack([params[f"Wroot_{l}_ba"], params[f"Wroot_{l}_ab"]])
        b_st = jnp.stack([params[f"brel_{l}_ba"].reshape(1, H),
                          params[f"brel_{l}_ab"].reshape(1, H)])
        hst = _dense(aggst, hst, wrel_st, wroot_st, b_st, l < 2)

    ga, gb = _pool(hst, batch_a3, batch_b3)
    return _mlp(ga, gb, params)


# speedup vs baseline: 8.7597x; 1.1524x over previous
import functools

import jax
import jax.numpy as jnp
from jax import lax
from jax.experimental import pallas as pl
from jax.experimental.pallas import tpu as pltpu
from jax.experimental.pallas import tpu_sc as plsc

H = 128
N = 10000
E = 320000
G = 64
NT = 16
CHUNK = 128
CPT = 160
E_PAD = CPT * CHUNK * NT
ACC_ROWS = N + 240
LANE = 16

_MESH = plsc.VectorSubcoreMesh(core_axis_name="c", subcore_axis_name="s")


@functools.partial(
    pl.kernel,
    mesh=_MESH,
    out_type=[
        jax.ShapeDtypeStruct((2, N, H), jnp.float32),
    ],
    scratch_types=[
        pltpu.VMEM_SHARED((ACC_ROWS, H), jnp.float32),
        pltpu.VMEM((64, 64), jnp.int32),
        pltpu.VMEM((64, 64), jnp.int32),
        pltpu.VMEM((64, H), jnp.float32),
        pltpu.VMEM((64, H), jnp.float32),
        pltpu.VMEM((64, H), jnp.float32),
        pltpu.VMEM((64, H), jnp.float32),
        pltpu.SemaphoreType.DMA,
        pltpu.SemaphoreType.DMA,
        pltpu.SemaphoreType.DMA,
        pltpu.SemaphoreType.DMA,
        pltpu.SemaphoreType.DMA,
        pltpu.SemaphoreType.DMA,
        pltpu.SemaphoreType.DMA,
        pltpu.SemaphoreType.DMA,
        pltpu.SemaphoreType.DMA,
    ],
)
def _hetero_agg(hst, src_ab, dst_ab, src_ba, dst_ba,
                agg_out,
                acc, src_v, dst_v, b0, b1, b2, b3,
                g0, g1, g2, g3, c0, c1, c2, c3, sem_i):
    c = lax.axis_index("c")
    s = lax.axis_index("s")
    BUFS = (b0, b1, b2, b3)
    GS = (g0, g1, g2, g3)
    CS = (c0, c1, c2, c3)

    def _zrow(i, _):
        def _zcol(j, _):
            b0[i, pl.ds(j * LANE, LANE)] = jnp.zeros((LANE,), jnp.float32)
            return 0
        return lax.fori_loop(0, H // LANE, _zcol, 0)
    lax.fori_loop(0, 64, _zrow, 0)

    def _zacc(k, _):
        pltpu.sync_copy(b0, acc.at[pl.ds(s * 640 + k * 64, 64)])
        return 0
    lax.fori_loop(0, 10, _zacc, 0)
    plsc.subcore_barrier()

    CPT_A = 2 * CPT
    NBLK = CPT_A // 32

    def _run(h, src2, dst2):
        base = s * CPT_A
        pltpu.sync_copy(src2.at[pl.ds(base, 32)], src_v.at[pl.ds(0, 32)])
        pltpu.sync_copy(dst2.at[pl.ds(base, 32)], dst_v.at[pl.ds(0, 32)])
        for v in range(3):
            pltpu.async_copy(h.at[src_v.at[v]], BUFS[v], GS[v])

        def _blk(o, _):
            rb = lax.rem(o, 2) * 32
            nrb = lax.rem(o + 1, 2) * 32

            @pl.when(o < NBLK - 1)
            def _():
                pltpu.async_copy(src2.at[pl.ds(base + (o + 1) * 32, 32)],
                                 src_v.at[pl.ds(nrb, 32)], sem_i)
                pltpu.async_copy(dst2.at[pl.ds(base + (o + 1) * 32, 32)],
                                 dst_v.at[pl.ds(nrb, 32)], sem_i)

            def _quad(q, _):
                for v in range(4):
                    kl = 4 * q + v
                    w = (v + 3) % 4

                    if v == 1:
                        @pl.when(jnp.logical_and(q == 7, o < NBLK - 1))
                        def _():
                            pltpu.make_async_copy(
                                src2.at[pl.ds(base + (o + 1) * 32, 32)],
                                src_v.at[pl.ds(nrb, 32)], sem_i).wait()
                            pltpu.make_async_copy(
                                dst2.at[pl.ds(base + (o + 1) * 32, 32)],
                                dst_v.at[pl.ds(nrb, 32)], sem_i).wait()

                    if v == 0:
                        row3 = rb + kl + 3
                        skip = jnp.bool_(False)
                        first = jnp.logical_and(o == 0, q == 0)
                    else:
                        row3 = jnp.where(q < 7, rb + kl + 3, nrb + v - 1)
                        skip = jnp.logical_and(q == 7, o == NBLK - 1)
                        first = jnp.bool_(False)

                    @pl.when(jnp.logical_not(jnp.logical_or(skip, first)))
                    def _(row3=row3, w=w):
                        pltpu.make_async_copy(BUFS[w], acc.at[dst_v.at[0]],
                                              CS[w]).wait()
                        pltpu.async_copy(h.at[src_v.at[row3]], BUFS[w], GS[w])

                    @pl.when(first)
                    def _(row3=row3, w=w):
                        pltpu.async_copy(h.at[src_v.at[row3]], BUFS[w], GS[w])

                    pltpu.make_async_copy(h.at[src_v.at[rb + kl]], BUFS[v],
                                          GS[v]).wait()
                    pltpu.async_copy(BUFS[v], acc.at[dst_v.at[rb + kl]],
                                     CS[v], add=True)
                return 0
            lax.fori_loop(0, 8, _quad, 0)
            return 0
        lax.fori_loop(0, NBLK, _blk, 0)
        for v in range(4):
            pltpu.make_async_copy(BUFS[v], acc.at[dst_v.at[0]], CS[v]).wait()

    @pl.when(c == 0)
    def _():
        _run(hst.at[0], src_ab, dst_ab)

    @pl.when(c != 0)
    def _():
        _run(hst.at[1], src_ba, dst_ba)

    plsc.subcore_barrier()

    def _copy_out(t):
        @pl.when(s < 15)
        def _():
            pltpu.sync_copy(acc.at[pl.ds(s * 640, 640)],
                            agg_out.at[t, pl.ds(s * 640, 640)])

        @pl.when(s == 15)
        def _():
            pltpu.sync_copy(acc.at[pl.ds(9600, 400)],
                            agg_out.at[t, pl.ds(9600, 400)])

    @pl.when(c == 0)
    def _():
        _copy_out(1)

    @pl.when(c != 0)
    def _():
        _copy_out(0)


ROWS_PT = 640
N_PAD_POOL = NT * ROWS_PT
TAB_ROWS = G + 8


@functools.partial(
    pl.kernel,
    mesh=_MESH,
    out_type=[
        jax.ShapeDtypeStruct((G, H), jnp.float32),
        jax.ShapeDtypeStruct((G, H), jnp.float32),
    ],
    scratch_types=[
        pltpu.VMEM_SHARED((NT, G, H), jnp.float32),
        pltpu.VMEM((ROWS_PT // CHUNK, CHUNK), jnp.int32),
        pltpu.VMEM((ROWS_PT, H), jnp.float32),
        pltpu.VMEM((TAB_ROWS, H), jnp.float32),
        pltpu.VMEM((NT, 8, H), jnp.float32),
    ],
)
def _pool(hst, batch_a3, batch_b3,
          ga_out, gb_out,
          part, bat_v, rows_v, tab_v, mrg_v):
    c = lax.axis_index("c")
    s = lax.axis_index("s")
    neg_inf = jnp.full((LANE,), -jnp.inf, jnp.float32)

    def _irow(i, _):
        def _icol(j, _):
            tab_v[i, pl.ds(j * LANE, LANE)] = neg_inf
            return 0
        return lax.fori_loop(0, H // LANE, _icol, 0)
    lax.fori_loop(0, TAB_ROWS, _irow, 0)

    def _run(t, bat3):
        pltpu.sync_copy(bat3.at[s], bat_v)
        @pl.when(s < 15)
        def _():
            pltpu.sync_copy(hst.at[t, pl.ds(s * ROWS_PT, ROWS_PT)], rows_v)

        @pl.when(s == 15)
        def _():
            pltpu.sync_copy(hst.at[t, pl.ds(15 * ROWS_PT, N - 15 * ROWS_PT)],
                            rows_v.at[pl.ds(0, N - 15 * ROWS_PT)])

        def _grp(g, carry):
            prev_bi = carry[0]
            runs = carry[1:]
            bvec = bat_v[g // 8, pl.ds((g % 8) * LANE, LANE)]
            for ii in range(LANE):
                bi = bvec[ii]
                i = g * LANE + ii
                changed = jnp.logical_and(bi != prev_bi, prev_bi >= 0)

                @pl.when(changed)
                def _(runs=runs, prev_bi=prev_bi):
                    for j in range(H // LANE):
                        cur = tab_v[prev_bi, pl.ds(j * LANE, LANE)]
                        tab_v[prev_bi, pl.ds(j * LANE, LANE)] = \
                            jnp.maximum(cur, runs[j])

                fresh = jnp.logical_or(changed, prev_bi < 0)
                runs = tuple(
                    jnp.maximum(jnp.where(fresh, neg_inf, runs[j]),
                                rows_v[i, pl.ds(j * LANE, LANE)])
                    for j in range(H // LANE))
                prev_bi = bi
            return (prev_bi,) + runs

        init = (jnp.int32(-1),) + tuple(neg_inf for _ in range(H // LANE))
        final = lax.fori_loop(0, ROWS_PT // LANE, _grp, init)
        last_bi = final[0]

        @pl.when(last_bi >= 0)
        def _():
            for j in range(H // LANE):
                cur = tab_v[last_bi, pl.ds(j * LANE, LANE)]
                tab_v[last_bi, pl.ds(j * LANE, LANE)] = \
                    jnp.maximum(cur, final[1 + j])

    @pl.when(c == 0)
    def _():
        _run(0, batch_a3)

    @pl.when(c != 0)
    def _():
        _run(1, batch_b3)

    pltpu.sync_copy(tab_v.at[pl.ds(0, G)], part.at[s])
    plsc.subcore_barrier()

    @pl.when(s < 8)
    def _():
        def _fetch(t, _):
            pltpu.sync_copy(part.at[t, pl.ds(s * 8, 8)], mrg_v.at[t])
            return 0
        lax.fori_loop(0, NT, _fetch, 0)

        def _red_t(t, _):
            def _red_g(g, _):
                def _red_j(j, _):
                    a = mrg_v[0, g, pl.ds(j * LANE, LANE)]
                    b = mrg_v[t, g, pl.ds(j * LANE, LANE)]
                    mrg_v[0, g, pl.ds(j * LANE, LANE)] = jnp.maximum(a, b)
                    return 0
                return lax.fori_loop(0, H // LANE, _red_j, 0)
            return lax.fori_loop(0, 8, _red_g, 0)
        lax.fori_loop(1, NT, _red_t, 0)

        @pl.when(c == 0)
        def _():
            pltpu.sync_copy(mrg_v.at[0], ga_out.at[pl.ds(s * 8, 8)])

        @pl.when(c != 0)
        def _():
            pltpu.sync_copy(mrg_v.at[0], gb_out.at[pl.ds(s * 8, 8)])


def _dense_body(relu, agg_ref, h_ref, wrel_ref, wroot_ref, b_ref, o_ref):
    y = (jnp.dot(agg_ref[0], wrel_ref[0], preferred_element_type=jnp.float32)
         + jnp.dot(h_ref[0], wroot_ref[0], preferred_element_type=jnp.float32)
         + b_ref[0])
    o_ref[0] = jnp.maximum(y, 0.0) if relu else y


def _dense(aggst, hst, wrel_st, wroot_st, b_st, relu):
    B = 1000
    return pl.pallas_call(
        functools.partial(_dense_body, relu),
        grid=(2, N // B),
        in_specs=[
            pl.BlockSpec((1, B, H), lambda t, i: (t, i, 0)),
            pl.BlockSpec((1, B, H), lambda t, i: (t, i, 0)),
            pl.BlockSpec((1, H, H), lambda t, i: (t, 0, 0)),
            pl.BlockSpec((1, H, H), lambda t, i: (t, 0, 0)),
            pl.BlockSpec((1, 1, H), lambda t, i: (t, 0, 0)),
        ],
        out_specs=pl.BlockSpec((1, B, H), lambda t, i: (t, i, 0)),
        out_shape=jax.ShapeDtypeStruct((2, N, H), jnp.float32),
    )(aggst, hst, wrel_st, wroot_st, b_st)


def _mlp_body(ga_ref, gb_ref, w1a, b1a, w2a, b2a, w1b, b1b, w2b, b2b,
              ow, obias, o_ref):
    oa = jnp.maximum(
        jnp.dot(ga_ref[...], w1a[...], preferred_element_type=jnp.float32)
        + b1a[...], 0.0)
    oa = jnp.dot(oa, w2a[...], preferred_element_type=jnp.float32) + b2a[...]
    ob = jnp.maximum(
        jnp.dot(gb_ref[...], w1b[...], preferred_element_type=jnp.float32)
        + b1b[...], 0.0)
    ob = jnp.dot(ob, w2b[...], preferred_element_type=jnp.float32) + b2b[...]
    o_ref[...] = oa * ow[0, 0] + ob * ow[1, 0] + obias[0, 0]


def _mlp(ga, gb, p):
    args = (ga, gb,
            p["mlpW1_a"], p["mlpb1_a"].reshape(1, 5),
            p["mlpW2_a"], p["mlpb2_a"].reshape(1, 1),
            p["mlpW1_b"], p["mlpb1_b"].reshape(1, 5),
            p["mlpW2_b"], p["mlpb2_b"].reshape(1, 1),
            p["outW"], p["outb"].reshape(1, 1))
    return pl.pallas_call(
        _mlp_body,
        out_shape=jax.ShapeDtypeStruct((G, 1), jnp.float32),
    )(*args)


def _prep_edges(ei):
    pad = E_PAD - E
    src_pad = (jnp.arange(pad, dtype=jnp.int32) * 97) % N
    dst_pad = N + (jnp.arange(pad, dtype=jnp.int32) % 128)
    src = jnp.concatenate([ei[0], src_pad]).reshape(E_PAD // 64, 64)
    dst = jnp.concatenate([ei[1], dst_pad]).reshape(E_PAD // 64, 64)
    return src, dst


def kernel(x_a, x_b, edge_index_ab, edge_index_ba, batch_a, batch_b, params):
    src_ab, dst_ab = _prep_edges(edge_index_ab)
    src_ba, dst_ba = _prep_edges(edge_index_ba)

    pad_n = N_PAD_POOL - N
    batch_a3 = jnp.concatenate(
        [batch_a, jnp.full((pad_n,), G, jnp.int32)]).reshape(NT, -1, CHUNK)
    batch_b3 = jnp.concatenate(
        [batch_b, jnp.full((pad_n,), G, jnp.int32)]).reshape(NT, -1, CHUNK)

    hst = jnp.stack([x_a, x_b])
    for l in range(3):
        (aggst,) = _hetero_agg(hst, src_ab, dst_ab, src_ba, dst_ba)
        wrel_st = jnp.stack([params[f"Wrel_{l}_ba"], params[f"Wrel_{l}_ab"]])
        wroot_st = jnp.stack([params[f"Wroot_{l}_ba"], params[f"Wroot_{l}_ab"]])
        b_st = jnp.stack([params[f"brel_{l}_ba"].reshape(1, H),
                          params[f"brel_{l}_ab"].reshape(1, H)])
        hst = _dense(aggst, hst, wrel_st, wroot_st, b_st, l < 2)

    ga, gb = _pool(hst, batch_a3, batch_b3)
    return _mlp(ga, gb, params)


# speedup vs baseline: 9.1960x; 1.0498x over previous
import functools

import jax
import jax.numpy as jnp
from jax import lax
from jax.experimental import pallas as pl
from jax.experimental.pallas import tpu as pltpu
from jax.experimental.pallas import tpu_sc as plsc

H = 128
N = 10000
E = 320000
G = 64
NT = 16
CHUNK = 128
CPT = 160
E_PAD = CPT * CHUNK * NT
ACC_ROWS = N + 240
LANE = 16

_MESH = plsc.VectorSubcoreMesh(core_axis_name="c", subcore_axis_name="s")


@functools.partial(
    pl.kernel,
    mesh=_MESH,
    out_type=[
        jax.ShapeDtypeStruct((2, N, H), jnp.float32),
    ],
    scratch_types=[
        pltpu.VMEM_SHARED((ACC_ROWS, H), jnp.float32),
        pltpu.VMEM((64, 64), jnp.int32),
        pltpu.VMEM((64, 64), jnp.int32),
        pltpu.VMEM((64, H), jnp.float32),
        pltpu.VMEM((64, H), jnp.float32),
        pltpu.VMEM((64, H), jnp.float32),
        pltpu.VMEM((64, H), jnp.float32),
        pltpu.SemaphoreType.DMA,
        pltpu.SemaphoreType.DMA,
        pltpu.SemaphoreType.DMA,
        pltpu.SemaphoreType.DMA,
        pltpu.SemaphoreType.DMA,
        pltpu.SemaphoreType.DMA,
        pltpu.SemaphoreType.DMA,
        pltpu.SemaphoreType.DMA,
        pltpu.SemaphoreType.DMA,
    ],
)
def _hetero_agg(h_a, h_b, src_ab, dst_ab, src_ba, dst_ba,
                agg_out,
                acc, src_v, dst_v, b0, b1, b2, b3,
                g0, g1, g2, g3, c0, c1, c2, c3, sem_i):
    c = lax.axis_index("c")
    s = lax.axis_index("s")
    BUFS = (b0, b1, b2, b3)
    GS = (g0, g1, g2, g3)
    CS = (c0, c1, c2, c3)

    def _zrow(i, _):
        def _zcol(j, _):
            b3[i, pl.ds(j * LANE, LANE)] = jnp.zeros((LANE,), jnp.float32)
            return 0
        return lax.fori_loop(0, H // LANE, _zcol, 0)
    lax.fori_loop(0, 64, _zrow, 0)

    CPT_A = 2 * CPT
    NBLK = CPT_A // 32

    def _run(h, src2, dst2):
        base = s * CPT_A
        pltpu.sync_copy(src2.at[pl.ds(base, 32)], src_v.at[pl.ds(0, 32)])
        pltpu.sync_copy(dst2.at[pl.ds(base, 32)], dst_v.at[pl.ds(0, 32)])
        for v in range(3):
            pltpu.async_copy(h.at[src_v.at[v]], BUFS[v], GS[v])

        def _zacc(k, _):
            pltpu.sync_copy(b3, acc.at[pl.ds(s * 640 + k * 64, 64)])
            return 0
        lax.fori_loop(0, 10, _zacc, 0)
        plsc.subcore_barrier()

        def _blk(o, _):
            rb = lax.rem(o, 2) * 32
            nrb = lax.rem(o + 1, 2) * 32

            @pl.when(o < NBLK - 1)
            def _():
                pltpu.async_copy(src2.at[pl.ds(base + (o + 1) * 32, 32)],
                                 src_v.at[pl.ds(nrb, 32)], sem_i)
                pltpu.async_copy(dst2.at[pl.ds(base + (o + 1) * 32, 32)],
                                 dst_v.at[pl.ds(nrb, 32)], sem_i)

            def _quad(q, _):
                for v in range(4):
                    kl = 4 * q + v
                    w = (v + 3) % 4

                    if v == 1:
                        @pl.when(jnp.logical_and(q == 7, o < NBLK - 1))
                        def _():
                            pltpu.make_async_copy(
                                src2.at[pl.ds(base + (o + 1) * 32, 32)],
                                src_v.at[pl.ds(nrb, 32)], sem_i).wait()
                            pltpu.make_async_copy(
                                dst2.at[pl.ds(base + (o + 1) * 32, 32)],
                                dst_v.at[pl.ds(nrb, 32)], sem_i).wait()

                    if v == 0:
                        row3 = rb + kl + 3
                        skip = jnp.bool_(False)
                        first = jnp.logical_and(o == 0, q == 0)
                    else:
                        row3 = jnp.where(q < 7, rb + kl + 3, nrb + v - 1)
                        skip = jnp.logical_and(q == 7, o == NBLK - 1)
                        first = jnp.bool_(False)

                    @pl.when(jnp.logical_not(jnp.logical_or(skip, first)))
                    def _(row3=row3, w=w):
                        pltpu.make_async_copy(BUFS[w], acc.at[dst_v.at[0]],
                                              CS[w]).wait()
                        pltpu.async_copy(h.at[src_v.at[row3]], BUFS[w], GS[w])

                    @pl.when(first)
                    def _(row3=row3, w=w):
                        pltpu.async_copy(h.at[src_v.at[row3]], BUFS[w], GS[w])

                    pltpu.make_async_copy(h.at[src_v.at[rb + kl]], BUFS[v],
                                          GS[v]).wait()
                    pltpu.async_copy(BUFS[v], acc.at[dst_v.at[rb + kl]],
                                     CS[v], add=True)
                return 0
            lax.fori_loop(0, 8, _quad, 0)
            return 0
        lax.fori_loop(0, NBLK, _blk, 0)
        for v in range(4):
            pltpu.make_async_copy(BUFS[v], acc.at[dst_v.at[0]], CS[v]).wait()

    @pl.when(c == 0)
    def _():
        _run(h_a, src_ab, dst_ab)

    @pl.when(c != 0)
    def _():
        _run(h_b, src_ba, dst_ba)

    plsc.subcore_barrier()

    def _copy_out(t):
        @pl.when(s < 15)
        def _():
            pltpu.sync_copy(acc.at[pl.ds(s * 640, 640)],
                            agg_out.at[t, pl.ds(s * 640, 640)])

        @pl.when(s == 15)
        def _():
            pltpu.sync_copy(acc.at[pl.ds(9600, 400)],
                            agg_out.at[t, pl.ds(9600, 400)])

    @pl.when(c == 0)
    def _():
        _copy_out(1)

    @pl.when(c != 0)
    def _():
        _copy_out(0)


ROWS_PT = 640
N_PAD_POOL = NT * ROWS_PT
TAB_ROWS = G + 8


@functools.partial(
    pl.kernel,
    mesh=_MESH,
    out_type=[
        jax.ShapeDtypeStruct((G, H), jnp.float32),
        jax.ShapeDtypeStruct((G, H), jnp.float32),
    ],
    scratch_types=[
        pltpu.VMEM_SHARED((NT, G, H), jnp.float32),
        pltpu.VMEM((ROWS_PT // CHUNK, CHUNK), jnp.int32),
        pltpu.VMEM((ROWS_PT, H), jnp.float32),
        pltpu.VMEM((TAB_ROWS, H), jnp.float32),
        pltpu.VMEM((NT, 8, H), jnp.float32),
    ],
)
def _pool(h_a, h_b, batch_a3, batch_b3,
          ga_out, gb_out,
          part, bat_v, rows_v, tab_v, mrg_v):
    c = lax.axis_index("c")
    s = lax.axis_index("s")
    neg_inf = jnp.full((LANE,), -jnp.inf, jnp.float32)

    def _irow(i, _):
        def _icol(j, _):
            tab_v[i, pl.ds(j * LANE, LANE)] = neg_inf
            return 0
        return lax.fori_loop(0, H // LANE, _icol, 0)
    lax.fori_loop(0, TAB_ROWS, _irow, 0)

    def _run(h, bat3):
        pltpu.sync_copy(bat3.at[s], bat_v)
        @pl.when(s < 15)
        def _():
            pltpu.sync_copy(h.at[pl.ds(s * ROWS_PT, ROWS_PT)], rows_v)

        @pl.when(s == 15)
        def _():
            pltpu.sync_copy(h.at[pl.ds(15 * ROWS_PT, N - 15 * ROWS_PT)],
                            rows_v.at[pl.ds(0, N - 15 * ROWS_PT)])

        def _grp(g, carry):
            prev_bi = carry[0]
            runs = carry[1:]
            bvec = bat_v[g // 8, pl.ds((g % 8) * LANE, LANE)]
            for ii in range(LANE):
                bi = bvec[ii]
                i = g * LANE + ii
                changed = jnp.logical_and(bi != prev_bi, prev_bi >= 0)

                @pl.when(changed)
                def _(runs=runs, prev_bi=prev_bi):
                    for j in range(H // LANE):
                        cur = tab_v[prev_bi, pl.ds(j * LANE, LANE)]
                        tab_v[prev_bi, pl.ds(j * LANE, LANE)] = \
                            jnp.maximum(cur, runs[j])

                fresh = jnp.logical_or(changed, prev_bi < 0)
                runs = tuple(
                    jnp.maximum(jnp.where(fresh, neg_inf, runs[j]),
                                rows_v[i, pl.ds(j * LANE, LANE)])
                    for j in range(H // LANE))
                prev_bi = bi
            return (prev_bi,) + runs

        init = (jnp.int32(-1),) + tuple(neg_inf for _ in range(H // LANE))
        final = lax.fori_loop(0, ROWS_PT // LANE, _grp, init)
        last_bi = final[0]

        @pl.when(last_bi >= 0)
        def _():
            for j in range(H // LANE):
                cur = tab_v[last_bi, pl.ds(j * LANE, LANE)]
                tab_v[last_bi, pl.ds(j * LANE, LANE)] = \
                    jnp.maximum(cur, final[1 + j])

    @pl.when(c == 0)
    def _():
        _run(h_a, batch_a3)

    @pl.when(c != 0)
    def _():
        _run(h_b, batch_b3)

    pltpu.sync_copy(tab_v.at[pl.ds(0, G)], part.at[s])
    plsc.subcore_barrier()

    @pl.when(s < 8)
    def _():
        def _fetch(t, _):
            pltpu.sync_copy(part.at[t, pl.ds(s * 8, 8)], mrg_v.at[t])
            return 0
        lax.fori_loop(0, NT, _fetch, 0)

        def _red_t(t, _):
            def _red_g(g, _):
                def _red_j(j, _):
                    a = mrg_v[0, g, pl.ds(j * LANE, LANE)]
                    b = mrg_v[t, g, pl.ds(j * LANE, LANE)]
                    mrg_v[0, g, pl.ds(j * LANE, LANE)] = jnp.maximum(a, b)
                    return 0
                return lax.fori_loop(0, H // LANE, _red_j, 0)
            return lax.fori_loop(0, 8, _red_g, 0)
        lax.fori_loop(1, NT, _red_t, 0)

        @pl.when(c == 0)
        def _():
            pltpu.sync_copy(mrg_v.at[0], ga_out.at[pl.ds(s * 8, 8)])

        @pl.when(c != 0)
        def _():
            pltpu.sync_copy(mrg_v.at[0], gb_out.at[pl.ds(s * 8, 8)])


def _dense_body(relu, agga_ref, aggb_ref, ha_ref, hb_ref,
                wrel_a, wroot_a, wrel_b, wroot_b, ba_ref, bb_ref,
                oa_ref, ob_ref):
    ya = (jnp.dot(agga_ref[0], wrel_a[...], preferred_element_type=jnp.float32)
          + jnp.dot(ha_ref[...], wroot_a[...],
                    preferred_element_type=jnp.float32)
          + ba_ref[...])
    yb = (jnp.dot(aggb_ref[0], wrel_b[...], preferred_element_type=jnp.float32)
          + jnp.dot(hb_ref[...], wroot_b[...],
                    preferred_element_type=jnp.float32)
          + bb_ref[...])
    oa_ref[...] = jnp.maximum(ya, 0.0) if relu else ya
    ob_ref[...] = jnp.maximum(yb, 0.0) if relu else yb


def _dense(aggst, h_a, h_b, wrel_a, wroot_a, wrel_b, wroot_b, b_a, b_b, relu):
    B = 1000
    return pl.pallas_call(
        functools.partial(_dense_body, relu),
        grid=(N // B,),
        in_specs=[
            pl.BlockSpec((1, B, H), lambda i: (0, i, 0)),
            pl.BlockSpec((1, B, H), lambda i: (1, i, 0)),
            pl.BlockSpec((B, H), lambda i: (i, 0)),
            pl.BlockSpec((B, H), lambda i: (i, 0)),
            pl.BlockSpec((H, H), lambda i: (0, 0)),
            pl.BlockSpec((H, H), lambda i: (0, 0)),
            pl.BlockSpec((H, H), lambda i: (0, 0)),
            pl.BlockSpec((H, H), lambda i: (0, 0)),
            pl.BlockSpec((1, H), lambda i: (0, 0)),
            pl.BlockSpec((1, H), lambda i: (0, 0)),
        ],
        out_specs=[
            pl.BlockSpec((B, H), lambda i: (i, 0)),
            pl.BlockSpec((B, H), lambda i: (i, 0)),
        ],
        out_shape=[
            jax.ShapeDtypeStruct((N, H), jnp.float32),
            jax.ShapeDtypeStruct((N, H), jnp.float32),
        ],
    )(aggst, aggst, h_a, h_b, wrel_a, wroot_a, wrel_b, wroot_b,
      b_a.reshape(1, H), b_b.reshape(1, H))


def _mlp_body(ga_ref, gb_ref, w1a, b1a, w2a, b2a, w1b, b1b, w2b, b2b,
              ow, obias, o_ref):
    oa = jnp.maximum(
        jnp.dot(ga_ref[...], w1a[...], preferred_element_type=jnp.float32)
        + b1a[...], 0.0)
    oa = jnp.dot(oa, w2a[...], preferred_element_type=jnp.float32) + b2a[...]
    ob = jnp.maximum(
        jnp.dot(gb_ref[...], w1b[...], preferred_element_type=jnp.float32)
        + b1b[...], 0.0)
    ob = jnp.dot(ob, w2b[...], preferred_element_type=jnp.float32) + b2b[...]
    o_ref[...] = oa * ow[0, 0] + ob * ow[1, 0] + obias[0, 0]


def _mlp(ga, gb, p):
    args = (ga, gb,
            p["mlpW1_a"], p["mlpb1_a"].reshape(1, 5),
            p["mlpW2_a"], p["mlpb2_a"].reshape(1, 1),
            p["mlpW1_b"], p["mlpb1_b"].reshape(1, 5),
            p["mlpW2_b"], p["mlpb2_b"].reshape(1, 1),
            p["outW"], p["outb"].reshape(1, 1))
    return pl.pallas_call(
        _mlp_body,
        out_shape=jax.ShapeDtypeStruct((G, 1), jnp.float32),
    )(*args)


def _prep_edges(ei):
    pad = E_PAD - E
    src_pad = (jnp.arange(pad, dtype=jnp.int32) * 97) % N
    dst_pad = N + (jnp.arange(pad, dtype=jnp.int32) % 128)
    src = jnp.concatenate([ei[0], src_pad]).reshape(E_PAD // 64, 64)
    dst = jnp.concatenate([ei[1], dst_pad]).reshape(E_PAD // 64, 64)
    return src, dst


def kernel(x_a, x_b, edge_index_ab, edge_index_ba, batch_a, batch_b, params):
    src_ab, dst_ab = _prep_edges(edge_index_ab)
    src_ba, dst_ba = _prep_edges(edge_index_ba)

    pad_n = N_PAD_POOL - N
    batch_a3 = jnp.concatenate(
        [batch_a, jnp.full((pad_n,), G, jnp.int32)]).reshape(NT, -1, CHUNK)
    batch_b3 = jnp.concatenate(
        [batch_b, jnp.full((pad_n,), G, jnp.int32)]).reshape(NT, -1, CHUNK)

    h_a, h_b = x_a, x_b
    for l in range(3):
        (aggst,) = _hetero_agg(h_a, h_b, src_ab, dst_ab, src_ba, dst_ba)
        h_a, h_b = _dense(aggst, h_a, h_b,
                          params[f"Wrel_{l}_ba"], params[f"Wroot_{l}_ba"],
                          params[f"Wrel_{l}_ab"], params[f"Wroot_{l}_ab"],
                          params[f"brel_{l}_ba"], params[f"brel_{l}_ab"],
                          l < 2)

    ga, gb = _pool(h_a, h_b, batch_a3, batch_b3)
    return _mlp(ga, gb, params)


# speedup vs baseline: 9.2137x; 1.0019x over previous
import functools

import jax
import jax.numpy as jnp
from jax import lax
from jax.experimental import pallas as pl
from jax.experimental.pallas import tpu as pltpu
from jax.experimental.pallas import tpu_sc as plsc

H = 128
N = 10000
E = 320000
G = 64
NT = 16
CHUNK = 128
CPT = 160
E_PAD = CPT * CHUNK * NT
ACC_ROWS = N + 240
LANE = 16

_MESH = plsc.VectorSubcoreMesh(core_axis_name="c", subcore_axis_name="s")


@functools.partial(
    pl.kernel,
    mesh=_MESH,
    out_type=[
        jax.ShapeDtypeStruct((2, N, H), jnp.float32),
    ],
    scratch_types=[
        pltpu.VMEM_SHARED((ACC_ROWS, H), jnp.float32),
        pltpu.VMEM((64, 64), jnp.int32),
        pltpu.VMEM((64, 64), jnp.int32),
        pltpu.VMEM((64, H), jnp.float32),
        pltpu.VMEM((64, H), jnp.float32),
        pltpu.VMEM((64, H), jnp.float32),
        pltpu.VMEM((64, H), jnp.float32),
        pltpu.SemaphoreType.DMA,
        pltpu.SemaphoreType.DMA,
        pltpu.SemaphoreType.DMA,
        pltpu.SemaphoreType.DMA,
        pltpu.SemaphoreType.DMA,
        pltpu.SemaphoreType.DMA,
        pltpu.SemaphoreType.DMA,
        pltpu.SemaphoreType.DMA,
        pltpu.SemaphoreType.DMA,
    ],
)
def _hetero_agg(h_a, h_b, src_ab, dst_ab, src_ba, dst_ba,
                agg_out,
                acc, src_v, dst_v, b0, b1, b2, b3,
                g0, g1, g2, g3, c0, c1, c2, c3, sem_i):
    c = lax.axis_index("c")
    s = lax.axis_index("s")
    BUFS = (b0, b1, b2, b3)
    GS = (g0, g1, g2, g3)
    CS = (c0, c1, c2, c3)

    def _zrow(i, _):
        def _zcol(j, _):
            b3[i, pl.ds(j * LANE, LANE)] = jnp.zeros((LANE,), jnp.float32)
            return 0
        return lax.fori_loop(0, H // LANE, _zcol, 0)
    lax.fori_loop(0, 64, _zrow, 0)

    CPT_A = 2 * CPT
    NBLK = CPT_A // 32

    base = s * CPT_A

    @pl.when(c == 0)
    def _():
        pltpu.sync_copy(src_ab.at[pl.ds(base, 32)], src_v.at[pl.ds(0, 32)])
        pltpu.sync_copy(dst_ab.at[pl.ds(base, 32)], dst_v.at[pl.ds(0, 32)])
        for v in range(3):
            pltpu.async_copy(h_a.at[src_v.at[v]], BUFS[v], GS[v])

    @pl.when(c != 0)
    def _():
        pltpu.sync_copy(src_ba.at[pl.ds(base, 32)], src_v.at[pl.ds(0, 32)])
        pltpu.sync_copy(dst_ba.at[pl.ds(base, 32)], dst_v.at[pl.ds(0, 32)])
        for v in range(3):
            pltpu.async_copy(h_b.at[src_v.at[v]], BUFS[v], GS[v])

    def _zacc(k, _):
        pltpu.sync_copy(b3, acc.at[pl.ds(s * 640 + k * 64, 64)])
        return 0
    lax.fori_loop(0, 10, _zacc, 0)
    plsc.subcore_barrier()

    def _run(h, src2, dst2):
        base = s * CPT_A

        def _blk(o, _):
            rb = lax.rem(o, 2) * 32
            nrb = lax.rem(o + 1, 2) * 32

            @pl.when(o < NBLK - 1)
            def _():
                pltpu.async_copy(src2.at[pl.ds(base + (o + 1) * 32, 32)],
                                 src_v.at[pl.ds(nrb, 32)], sem_i)
                pltpu.async_copy(dst2.at[pl.ds(base + (o + 1) * 32, 32)],
                                 dst_v.at[pl.ds(nrb, 32)], sem_i)

            def _quad(q, _):
                for v in range(4):
                    kl = 4 * q + v
                    w = (v + 3) % 4

                    if v == 1:
                        @pl.when(jnp.logical_and(q == 7, o < NBLK - 1))
                        def _():
                            pltpu.make_async_copy(
                                src2.at[pl.ds(base + (o + 1) * 32, 32)],
                                src_v.at[pl.ds(nrb, 32)], sem_i).wait()
                            pltpu.make_async_copy(
                                dst2.at[pl.ds(base + (o + 1) * 32, 32)],
                                dst_v.at[pl.ds(nrb, 32)], sem_i).wait()

                    if v == 0:
                        row3 = rb + kl + 3
                        skip = jnp.bool_(False)
                        first = jnp.logical_and(o == 0, q == 0)
                    else:
                        row3 = jnp.where(q < 7, rb + kl + 3, nrb + v - 1)
                        skip = jnp.logical_and(q == 7, o == NBLK - 1)
                        first = jnp.bool_(False)

                    @pl.when(jnp.logical_not(jnp.logical_or(skip, first)))
                    def _(row3=row3, w=w):
                        pltpu.make_async_copy(BUFS[w], acc.at[dst_v.at[0]],
                                              CS[w]).wait()
                        pltpu.async_copy(h.at[src_v.at[row3]], BUFS[w], GS[w])

                    @pl.when(first)
                    def _(row3=row3, w=w):
                        pltpu.async_copy(h.at[src_v.at[row3]], BUFS[w], GS[w])

                    pltpu.make_async_copy(h.at[src_v.at[rb + kl]], BUFS[v],
                                          GS[v]).wait()
                    pltpu.async_copy(BUFS[v], acc.at[dst_v.at[rb + kl]],
                                     CS[v], add=True)
                return 0
            lax.fori_loop(0, 8, _quad, 0)
            return 0
        lax.fori_loop(0, NBLK, _blk, 0)
        for v in range(4):
            pltpu.make_async_copy(BUFS[v], acc.at[dst_v.at[0]], CS[v]).wait()

    @pl.when(c == 0)
    def _():
        _run(h_a, src_ab, dst_ab)

    @pl.when(c != 0)
    def _():
        _run(h_b, src_ba, dst_ba)

    plsc.subcore_barrier()

    def _copy_out(t):
        @pl.when(s < 15)
        def _():
            pltpu.sync_copy(acc.at[pl.ds(s * 640, 640)],
                            agg_out.at[t, pl.ds(s * 640, 640)])

        @pl.when(s == 15)
        def _():
            pltpu.sync_copy(acc.at[pl.ds(9600, 400)],
                            agg_out.at[t, pl.ds(9600, 400)])

    @pl.when(c == 0)
    def _():
        _copy_out(1)

    @pl.when(c != 0)
    def _():
        _copy_out(0)


ROWS_PT = 640
N_PAD_POOL = NT * ROWS_PT
TAB_ROWS = G + 8


@functools.partial(
    pl.kernel,
    mesh=_MESH,
    out_type=[
        jax.ShapeDtypeStruct((G, H), jnp.float32),
        jax.ShapeDtypeStruct((G, H), jnp.float32),
    ],
    scratch_types=[
        pltpu.VMEM_SHARED((NT, G, H), jnp.float32),
        pltpu.VMEM((ROWS_PT // CHUNK, CHUNK), jnp.int32),
        pltpu.VMEM((ROWS_PT, H), jnp.float32),
        pltpu.VMEM((TAB_ROWS, H), jnp.float32),
        pltpu.VMEM((NT, 8, H), jnp.float32),
    ],
)
def _pool(h_a, h_b, batch_a3, batch_b3,
          ga_out, gb_out,
          part, bat_v, rows_v, tab_v, mrg_v):
    c = lax.axis_index("c")
    s = lax.axis_index("s")
    neg_inf = jnp.full((LANE,), -jnp.inf, jnp.float32)

    def _irow(i, _):
        def _icol(j, _):
            tab_v[i, pl.ds(j * LANE, LANE)] = neg_inf
            return 0
        return lax.fori_loop(0, H // LANE, _icol, 0)
    lax.fori_loop(0, TAB_ROWS, _irow, 0)

    def _run(h, bat3):
        pltpu.sync_copy(bat3.at[s], bat_v)
        @pl.when(s < 15)
        def _():
            pltpu.sync_copy(h.at[pl.ds(s * ROWS_PT, ROWS_PT)], rows_v)

        @pl.when(s == 15)
        def _():
            pltpu.sync_copy(h.at[pl.ds(15 * ROWS_PT, N - 15 * ROWS_PT)],
                            rows_v.at[pl.ds(0, N - 15 * ROWS_PT)])

        def _grp(g, carry):
            prev_bi = carry[0]
            runs = carry[1:]
            bvec = bat_v[g // 8, pl.ds((g % 8) * LANE, LANE)]
            for ii in range(LANE):
                bi = bvec[ii]
                i = g * LANE + ii
                changed = jnp.logical_and(bi != prev_bi, prev_bi >= 0)

                @pl.when(changed)
                def _(runs=runs, prev_bi=prev_bi):
                    for j in range(H // LANE):
                        cur = tab_v[prev_bi, pl.ds(j * LANE, LANE)]
                        tab_v[prev_bi, pl.ds(j * LANE, LANE)] = \
                            jnp.maximum(cur, runs[j])

                fresh = jnp.logical_or(changed, prev_bi < 0)
                runs = tuple(
                    jnp.maximum(jnp.where(fresh, neg_inf, runs[j]),
                                rows_v[i, pl.ds(j * LANE, LANE)])
                    for j in range(H // LANE))
                prev_bi = bi
            return (prev_bi,) + runs

        init = (jnp.int32(-1),) + tuple(neg_inf for _ in range(H // LANE))
        final = lax.fori_loop(0, ROWS_PT // LANE, _grp, init)
        last_bi = final[0]

        @pl.when(last_bi >= 0)
        def _():
            for j in range(H // LANE):
                cur = tab_v[last_bi, pl.ds(j * LANE, LANE)]
                tab_v[last_bi, pl.ds(j * LANE, LANE)] = \
                    jnp.maximum(cur, final[1 + j])

    @pl.when(c == 0)
    def _():
        _run(h_a, batch_a3)

    @pl.when(c != 0)
    def _():
        _run(h_b, batch_b3)

    pltpu.sync_copy(tab_v.at[pl.ds(0, G)], part.at[s])
    plsc.subcore_barrier()

    @pl.when(s < 8)
    def _():
        def _fetch(t, _):
            pltpu.sync_copy(part.at[t, pl.ds(s * 8, 8)], mrg_v.at[t])
            return 0
        lax.fori_loop(0, NT, _fetch, 0)

        def _red_t(t, _):
            def _red_g(g, _):
                def _red_j(j, _):
                    a = mrg_v[0, g, pl.ds(j * LANE, LANE)]
                    b = mrg_v[t, g, pl.ds(j * LANE, LANE)]
                    mrg_v[0, g, pl.ds(j * LANE, LANE)] = jnp.maximum(a, b)
                    return 0
                return lax.fori_loop(0, H // LANE, _red_j, 0)
            return lax.fori_loop(0, 8, _red_g, 0)
        lax.fori_loop(1, NT, _red_t, 0)

        @pl.when(c == 0)
        def _():
            pltpu.sync_copy(mrg_v.at[0], ga_out.at[pl.ds(s * 8, 8)])

        @pl.when(c != 0)
        def _():
            pltpu.sync_copy(mrg_v.at[0], gb_out.at[pl.ds(s * 8, 8)])


def _dense_body(relu, agga_ref, aggb_ref, ha_ref, hb_ref,
                wrel_a, wroot_a, wrel_b, wroot_b, ba_ref, bb_ref,
                oa_ref, ob_ref):
    ya = (jnp.dot(agga_ref[0], wrel_a[...], preferred_element_type=jnp.float32)
          + jnp.dot(ha_ref[...], wroot_a[...],
                    preferred_element_type=jnp.float32)
          + ba_ref[...])
    yb = (jnp.dot(aggb_ref[0], wrel_b[...], preferred_element_type=jnp.float32)
          + jnp.dot(hb_ref[...], wroot_b[...],
                    preferred_element_type=jnp.float32)
          + bb_ref[...])
    oa_ref[...] = jnp.maximum(ya, 0.0) if relu else ya
    ob_ref[...] = jnp.maximum(yb, 0.0) if relu else yb


def _dense(aggst, h_a, h_b, wrel_a, wroot_a, wrel_b, wroot_b, b_a, b_b, relu):
    B = 1000
    return pl.pallas_call(
        functools.partial(_dense_body, relu),
        grid=(N // B,),
        in_specs=[
            pl.BlockSpec((1, B, H), lambda i: (0, i, 0)),
            pl.BlockSpec((1, B, H), lambda i: (1, i, 0)),
            pl.BlockSpec((B, H), lambda i: (i, 0)),
            pl.BlockSpec((B, H), lambda i: (i, 0)),
            pl.BlockSpec((H, H), lambda i: (0, 0)),
            pl.BlockSpec((H, H), lambda i: (0, 0)),
            pl.BlockSpec((H, H), lambda i: (0, 0)),
            pl.BlockSpec((H, H), lambda i: (0, 0)),
            pl.BlockSpec((1, H), lambda i: (0, 0)),
            pl.BlockSpec((1, H), lambda i: (0, 0)),
        ],
        out_specs=[
            pl.BlockSpec((B, H), lambda i: (i, 0)),
            pl.BlockSpec((B, H), lambda i: (i, 0)),
        ],
        out_shape=[
            jax.ShapeDtypeStruct((N, H), jnp.float32),
            jax.ShapeDtypeStruct((N, H), jnp.float32),
        ],
    )(aggst, aggst, h_a, h_b, wrel_a, wroot_a, wrel_b, wroot_b,
      b_a.reshape(1, H), b_b.reshape(1, H))


def _mlp_body(ga_ref, gb_ref, w1a, b1a, w2a, b2a, w1b, b1b, w2b, b2b,
              ow, obias, o_ref):
    oa = jnp.maximum(
        jnp.dot(ga_ref[...], w1a[...], preferred_element_type=jnp.float32)
        + b1a[...], 0.0)
    oa = jnp.dot(oa, w2a[...], preferred_element_type=jnp.float32) + b2a[...]
    ob = jnp.maximum(
        jnp.dot(gb_ref[...], w1b[...], preferred_element_type=jnp.float32)
        + b1b[...], 0.0)
    ob = jnp.dot(ob, w2b[...], preferred_element_type=jnp.float32) + b2b[...]
    o_ref[...] = oa * ow[0, 0] + ob * ow[1, 0] + obias[0, 0]


def _mlp(ga, gb, p):
    args = (ga, gb,
            p["mlpW1_a"], p["mlpb1_a"].reshape(1, 5),
            p["mlpW2_a"], p["mlpb2_a"].reshape(1, 1),
            p["mlpW1_b"], p["mlpb1_b"].reshape(1, 5),
            p["mlpW2_b"], p["mlpb2_b"].reshape(1, 1),
            p["outW"], p["outb"].reshape(1, 1))
    return pl.pallas_call(
        _mlp_body,
        out_shape=jax.ShapeDtypeStruct((G, 1), jnp.float32),
    )(*args)


def _prep_edges(ei):
    pad = E_PAD - E
    src_pad = (jnp.arange(pad, dtype=jnp.int32) * 97) % N
    dst_pad = N + (jnp.arange(pad, dtype=jnp.int32) % 128)
    src = jnp.concatenate([ei[0], src_pad]).reshape(E_PAD // 64, 64)
    dst = jnp.concatenate([ei[1], dst_pad]).reshape(E_PAD // 64, 64)
    return src, dst


def kernel(x_a, x_b, edge_index_ab, edge_index_ba, batch_a, batch_b, params):
    src_ab, dst_ab = _prep_edges(edge_index_ab)
    src_ba, dst_ba = _prep_edges(edge_index_ba)

    pad_n = N_PAD_POOL - N
    batch_a3 = jnp.concatenate(
        [batch_a, jnp.full((pad_n,), G, jnp.int32)]).reshape(NT, -1, CHUNK)
    batch_b3 = jnp.concatenate(
        [batch_b, jnp.full((pad_n,), G, jnp.int32)]).reshape(NT, -1, CHUNK)

    h_a, h_b = x_a, x_b
    for l in range(3):
        (aggst,) = _hetero_agg(h_a, h_b, src_ab, dst_ab, src_ba, dst_ba)
        h_a, h_b = _dense(aggst, h_a, h_b,
                          params[f"Wrel_{l}_ba"], params[f"Wroot_{l}_ba"],
                          params[f"Wrel_{l}_ab"], params[f"Wroot_{l}_ab"],
                          params[f"brel_{l}_ba"], params[f"brel_{l}_ab"],
                          l < 2)

    ga, gb = _pool(h_a, h_b, batch_a3, batch_b3)
    return _mlp(ga, gb, params)
